# baseline (speedup 1.0000x reference)
import functools

import jax
import jax.numpy as jnp
import numpy as np
from jax import lax
from jax.experimental import pallas as pl
from jax.experimental.pallas import tpu as pltpu
from jax.experimental.pallas import tpu_sc as plsc

N_NODES = 10000
N_EDGES = 160000
IN_DIM = 32
OUT_DIM = 32
NEF = 16
HID = 16
ALPHA = float(1.0 / np.sqrt(IN_DIM * 1))
EPS = float(jnp.finfo(jnp.float32).eps)

NC = 2
NS = 16
NW = NC * NS
EW = N_EDGES // NW
GC = 1000
NCHUNK = EW // GC
STRIPE = N_NODES // NS

@functools.lru_cache(maxsize=None)
def _get_mesh():
    return plsc.VectorSubcoreMesh(core_axis_name="c", subcore_axis_name="s",
                                  num_cores=NC, num_subcores=NS)


GCP = GC + 8
NG16 = GCP // 16


def _transpose_to_slabs(rows_v, bufT, iota16):
    def u_body(u, _):
        col_u = jnp.full((16,), u, dtype=jnp.int32)
        for g in range(NG16):
            row_idx = iota16 + (g * 16)
            vec = plsc.load_gather(rows_v, [row_idx, col_u])
            plsc.store_scatter(bufT, [col_u, row_idx], vec)
        return 0

    lax.fori_loop(0, IN_DIM, u_body, 0)


def _sc_gather_body(node_hbm, dst_hbm, o0, o1, o2, o3,
                    idx_v, rows_v, bufT, sem):
    wid = lax.axis_index("s") * NC + lax.axis_index("c")
    iota16 = lax.iota(jnp.int32, 16)
    outs = (o0, o1, o2, o3)
    for i in range(NCHUNK):
        base = wid * EW + i * GC
        pltpu.sync_copy(dst_hbm.at[pl.ds(base, GC)], idx_v)
        pltpu.async_copy(node_hbm.at[idx_v], rows_v.at[pl.ds(0, GC)],
                         sem).wait()
        _transpose_to_slabs(rows_v, bufT, iota16)
        for s in range(4):
            pltpu.sync_copy(bufT.at[pl.ds(8 * s, 8), pl.ds(0, GC)],
                            outs[s].at[:, pl.ds(base, GC)])


@functools.lru_cache(maxsize=None)
def _sc_gather():
    return pl.kernel(
        _sc_gather_body,
        out_type=tuple(
            jax.ShapeDtypeStruct((8, N_EDGES), jnp.float32)
            for _ in range(4)),
        mesh=_get_mesh(),
        scratch_types=[
            pltpu.VMEM((GC,), jnp.int32),
            pltpu.VMEM((GCP, IN_DIM), jnp.float32),
            pltpu.VMEM((IN_DIM, GCP), jnp.float32),
            pltpu.SemaphoreType.DMA,
        ],
        compiler_params=pltpu.CompilerParams(use_tc_tiling_on_sc=False, needs_layout_passes=False),
    )


def _transpose_from_slabs(sbuf, val_v, iota16):
    def u_body(u, _):
        col_u = jnp.full((16,), u, dtype=jnp.int32)
        for g in range(NG16):
            row_idx = iota16 + (g * 16)
            vec = plsc.load_gather(sbuf, [col_u, row_idx])
            plsc.store_scatter(val_v, [row_idx, col_u], vec)
        return 0

    lax.fori_loop(0, IN_DIM, u_body, 0)


def _sc_scatter_body(s0, s1, s2, s3, src_hbm, z32_hbm, z16_hbm, ones_hbm,
                     psum_hbm, pcnt_hbm,
                     idx_v, sbuf, val_v, ones_v, shared_sum, shared_cnt):
    cid = lax.axis_index("c")
    sid = lax.axis_index("s")
    iota16 = lax.iota(jnp.int32, 16)
    slabs = (s0, s1, s2, s3)
    row0 = sid * STRIPE
    pltpu.sync_copy(z32_hbm.at[pl.ds(row0, STRIPE)],
                    shared_sum.at[pl.ds(row0, STRIPE)])
    pltpu.sync_copy(z16_hbm.at[pl.ds(row0, STRIPE)],
                    shared_cnt.at[pl.ds(row0, STRIPE)])
    pltpu.sync_copy(ones_hbm, ones_v)
    plsc.subcore_barrier()
    wid = sid * NC + cid
    for i in range(NCHUNK):
        base = wid * EW + i * GC
        pltpu.sync_copy(src_hbm.at[pl.ds(base, GC)], idx_v)
        for s in range(4):
            pltpu.sync_copy(slabs[s].at[:, pl.ds(base, GC)],
                            sbuf.at[pl.ds(8 * s, 8), pl.ds(0, GC)])
        _transpose_from_slabs(sbuf, val_v, iota16)
        pltpu.sync_copy(val_v.at[pl.ds(0, GC)], shared_sum.at[idx_v],
                        add=True)
        pltpu.sync_copy(ones_v, shared_cnt.at[idx_v], add=True)
    plsc.subcore_barrier()
    pltpu.sync_copy(shared_sum.at[pl.ds(row0, STRIPE)],
                    psum_hbm.at[cid, pl.ds(row0, STRIPE)])
    pltpu.sync_copy(shared_cnt.at[pl.ds(row0, STRIPE)],
                    pcnt_hbm.at[cid, pl.ds(row0, STRIPE)])


@functools.lru_cache(maxsize=None)
def _sc_scatter():
    return pl.kernel(
        _sc_scatter_body,
        out_type=(
            jax.ShapeDtypeStruct((NC, N_NODES, OUT_DIM), jnp.float32),
            jax.ShapeDtypeStruct((NC, N_NODES, HID), jnp.float32),
        ),
        mesh=_get_mesh(),
        scratch_types=[
            pltpu.VMEM((GC,), jnp.int32),
            pltpu.VMEM((IN_DIM, GCP), jnp.float32),
            pltpu.VMEM((GCP, OUT_DIM), jnp.float32),
            pltpu.VMEM((GC, HID), jnp.float32),
            pltpu.VMEM_SHARED((N_NODES, OUT_DIM), jnp.float32),
            pltpu.VMEM_SHARED((N_NODES, HID), jnp.float32),
        ],
        compiler_params=pltpu.CompilerParams(use_tc_tiling_on_sc=False, needs_layout_passes=False),
    )


EB = 3200


def _dense_body(eaT_ref, x0_ref, x1_ref, x2_ref, x3_ref, shT_ref, w1_ref,
                b1c_ref, w2mT_ref, b2rT_ref,
                o0_ref, o1_ref, o2_ref, o3_ref):
    hT = jnp.maximum(
        jnp.dot(w1_ref[...], eaT_ref[...],
                preferred_element_type=jnp.float32) + b1c_ref[...], 0.0)
    xdT = jnp.concatenate(
        [x0_ref[...], x1_ref[...], x2_ref[...], x3_ref[...]], axis=0)
    xdsT = shT_ref[...] * xdT
    gT = jnp.dot(w2mT_ref[...], xdsT, preferred_element_type=jnp.float32)
    acc = jnp.dot(b2rT_ref[...], xdsT, preferred_element_type=jnp.float32)
    for j in range(HID):
        acc = acc + gT[j * OUT_DIM:(j + 1) * OUT_DIM, :] * hT[j:j + 1, :]
    o0_ref[...] = acc[0:8, :]
    o1_ref[...] = acc[8:16, :]
    o2_ref[...] = acc[16:24, :]
    o3_ref[...] = acc[24:32, :]


def _dense(eaT, xs, shT, w1, b1c, w2mT, b2rT):
    slab_spec = pl.BlockSpec((8, EB), lambda i: (0, i))
    return pl.pallas_call(
        _dense_body,
        grid=(N_EDGES // EB,),
        in_specs=[
            pl.BlockSpec((NEF, EB), lambda i: (0, i)),
            slab_spec, slab_spec, slab_spec, slab_spec,
            pl.BlockSpec((1, EB), lambda i: (0, i)),
            pl.BlockSpec((NEF, NEF), lambda i: (0, 0)),
            pl.BlockSpec((HID, 1), lambda i: (0, 0)),
            pl.BlockSpec((HID * OUT_DIM, IN_DIM), lambda i: (0, 0)),
            pl.BlockSpec((OUT_DIM, IN_DIM), lambda i: (0, 0)),
        ],
        out_specs=[slab_spec] * 4,
        out_shape=[jax.ShapeDtypeStruct((8, N_EDGES), jnp.float32)] * 4,
    )(eaT, *xs, shT, w1, b1c, w2mT, b2rT)


def _final_body(p_ref, c_ref, na_ref, out_ref):
    s = p_ref[0] + p_ref[1]
    cnt = c_ref[0, :, 0:1] + c_ref[1, :, 0:1]
    out_ref[...] = s / jnp.maximum(cnt, EPS) + na_ref[...]


def _final(psum, pcnt, node_attr):
    return pl.pallas_call(
        _final_body,
        out_shape=jax.ShapeDtypeStruct((N_NODES, OUT_DIM), jnp.float32),
    )(psum, pcnt, node_attr)


def kernel(node_attr, edge_index, edge_attr, edge_sh, fc_w1, fc_b1, fc_w2,
           fc_b2):
    src = edge_index[0]
    dst = edge_index[1]
    xs = _sc_gather()(node_attr, dst)
    b1c = fc_b1.reshape(HID, 1)
    w2mT = ALPHA * fc_w2.reshape(IN_DIM, OUT_DIM, HID).transpose(2, 1, 0)\
        .reshape(HID * OUT_DIM, IN_DIM)
    b2rT = ALPHA * fc_b2.reshape(IN_DIM, OUT_DIM).T
    ss = _dense(edge_attr.T, xs, edge_sh.T, fc_w1, b1c, w2mT, b2rT)
    z32 = jnp.zeros((N_NODES, OUT_DIM), jnp.float32)
    z16 = jnp.zeros((N_NODES, HID), jnp.float32)
    ones = jnp.ones((GC, HID), jnp.float32)
    psum, pcnt = _sc_scatter()(*ss, src, z32, z16, ones)
    return _final(psum, pcnt, node_attr)


# speedup vs baseline: 1.2232x; 1.2232x over previous
import functools

import jax
import jax.numpy as jnp
import numpy as np
from jax import lax
from jax.experimental import pallas as pl
from jax.experimental.pallas import tpu as pltpu
from jax.experimental.pallas import tpu_sc as plsc

N_NODES = 10000
N_EDGES = 160000
IN_DIM = 32
OUT_DIM = 32
NEF = 16
HID = 16
ALPHA = float(1.0 / np.sqrt(IN_DIM * 1))
EPS = float(jnp.finfo(jnp.float32).eps)

NC = 2
NS = 16
NW = NC * NS
EW = N_EDGES // NW
GC = 1000
NCHUNK = EW // GC
STRIPE = N_NODES // NS

@functools.lru_cache(maxsize=None)
def _get_mesh():
    return plsc.VectorSubcoreMesh(core_axis_name="c", subcore_axis_name="s",
                                  num_cores=NC, num_subcores=NS)


GCP = GC + 8
NG16 = GCP // 16


def _transpose_to_slabs(rows_v, bufT, iota16, col_us):
    @plsc.parallel_loop(0, NG16, unroll=4)
    def g_body(g):
        row_idx = iota16 + g * 16
        for u in range(IN_DIM):
            vec = plsc.load_gather(rows_v, [row_idx, col_us[u]])
            plsc.store_scatter(bufT, [col_us[u], row_idx], vec)


def _sc_gather_body(node_hbm, dst_hbm, o0, o1, o2, o3,
                    idx_v, rows_v, bufT, sem):
    wid = lax.axis_index("s") * NC + lax.axis_index("c")
    iota16 = lax.iota(jnp.int32, 16)
    col_us = [jnp.full((16,), u, dtype=jnp.int32) for u in range(IN_DIM)]
    outs = (o0, o1, o2, o3)
    for i in range(NCHUNK):
        base = wid * EW + i * GC
        pltpu.sync_copy(dst_hbm.at[pl.ds(base, GC)], idx_v)
        pltpu.async_copy(node_hbm.at[idx_v], rows_v.at[pl.ds(0, GC)],
                         sem).wait()
        _transpose_to_slabs(rows_v, bufT, iota16, col_us)
        for s in range(4):
            pltpu.sync_copy(bufT.at[pl.ds(8 * s, 8), pl.ds(0, GC)],
                            outs[s].at[:, pl.ds(base, GC)])


@functools.lru_cache(maxsize=None)
def _sc_gather():
    return pl.kernel(
        _sc_gather_body,
        out_type=tuple(
            jax.ShapeDtypeStruct((8, N_EDGES), jnp.float32)
            for _ in range(4)),
        mesh=_get_mesh(),
        scratch_types=[
            pltpu.VMEM((GC,), jnp.int32),
            pltpu.VMEM((GCP, IN_DIM), jnp.float32),
            pltpu.VMEM((IN_DIM, GCP), jnp.float32),
            pltpu.SemaphoreType.DMA,
        ],
        compiler_params=pltpu.CompilerParams(use_tc_tiling_on_sc=False, needs_layout_passes=False),
    )


def _transpose_from_slabs(sbuf, val_v, iota16, col_us):
    @plsc.parallel_loop(0, NG16, unroll=4)
    def g_body(g):
        row_idx = iota16 + g * 16
        for u in range(IN_DIM):
            vec = plsc.load_gather(sbuf, [col_us[u], row_idx])
            plsc.store_scatter(val_v, [row_idx, col_us[u]], vec)


def _sc_scatter_body(s0, s1, s2, s3, src_hbm, z32_hbm, z16_hbm, ones_hbm,
                     psum_hbm, pcnt_hbm,
                     idx_v, sbuf, val_v, ones_v, shared_sum, shared_cnt):
    cid = lax.axis_index("c")
    sid = lax.axis_index("s")
    iota16 = lax.iota(jnp.int32, 16)
    col_us = [jnp.full((16,), u, dtype=jnp.int32) for u in range(IN_DIM)]
    slabs = (s0, s1, s2, s3)
    row0 = sid * STRIPE
    pltpu.sync_copy(z32_hbm.at[pl.ds(row0, STRIPE)],
                    shared_sum.at[pl.ds(row0, STRIPE)])
    pltpu.sync_copy(z16_hbm.at[pl.ds(row0, STRIPE)],
                    shared_cnt.at[pl.ds(row0, STRIPE)])
    pltpu.sync_copy(ones_hbm, ones_v)
    plsc.subcore_barrier()
    wid = sid * NC + cid
    for i in range(NCHUNK):
        base = wid * EW + i * GC
        pltpu.sync_copy(src_hbm.at[pl.ds(base, GC)], idx_v)
        for s in range(4):
            pltpu.sync_copy(slabs[s].at[:, pl.ds(base, GC)],
                            sbuf.at[pl.ds(8 * s, 8), pl.ds(0, GC)])
        _transpose_from_slabs(sbuf, val_v, iota16, col_us)
        pltpu.sync_copy(val_v.at[pl.ds(0, GC)], shared_sum.at[idx_v],
                        add=True)
        pltpu.sync_copy(ones_v, shared_cnt.at[idx_v], add=True)
    plsc.subcore_barrier()
    pltpu.sync_copy(shared_sum.at[pl.ds(row0, STRIPE)],
                    psum_hbm.at[cid, pl.ds(row0, STRIPE)])
    pltpu.sync_copy(shared_cnt.at[pl.ds(row0, STRIPE)],
                    pcnt_hbm.at[cid, pl.ds(row0, STRIPE)])


@functools.lru_cache(maxsize=None)
def _sc_scatter():
    return pl.kernel(
        _sc_scatter_body,
        out_type=(
            jax.ShapeDtypeStruct((NC, N_NODES, OUT_DIM), jnp.float32),
            jax.ShapeDtypeStruct((NC, N_NODES, HID), jnp.float32),
        ),
        mesh=_get_mesh(),
        scratch_types=[
            pltpu.VMEM((GC,), jnp.int32),
            pltpu.VMEM((IN_DIM, GCP), jnp.float32),
            pltpu.VMEM((GCP, OUT_DIM), jnp.float32),
            pltpu.VMEM((GC, HID), jnp.float32),
            pltpu.VMEM_SHARED((N_NODES, OUT_DIM), jnp.float32),
            pltpu.VMEM_SHARED((N_NODES, HID), jnp.float32),
        ],
        compiler_params=pltpu.CompilerParams(use_tc_tiling_on_sc=False, needs_layout_passes=False),
    )


EB = 3200


def _dense_body(eaT_ref, x0_ref, x1_ref, x2_ref, x3_ref, shT_ref, w1_ref,
                b1c_ref, w2mT_ref, b2rT_ref,
                o0_ref, o1_ref, o2_ref, o3_ref):
    hT = jnp.maximum(
        jnp.dot(w1_ref[...], eaT_ref[...],
                preferred_element_type=jnp.float32) + b1c_ref[...], 0.0)
    xdT = jnp.concatenate(
        [x0_ref[...], x1_ref[...], x2_ref[...], x3_ref[...]], axis=0)
    xdsT = shT_ref[...] * xdT
    gT = jnp.dot(w2mT_ref[...], xdsT, preferred_element_type=jnp.float32)
    acc = jnp.dot(b2rT_ref[...], xdsT, preferred_element_type=jnp.float32)
    for j in range(HID):
        acc = acc + gT[j * OUT_DIM:(j + 1) * OUT_DIM, :] * hT[j:j + 1, :]
    o0_ref[...] = acc[0:8, :]
    o1_ref[...] = acc[8:16, :]
    o2_ref[...] = acc[16:24, :]
    o3_ref[...] = acc[24:32, :]


def _dense(eaT, xs, shT, w1, b1c, w2mT, b2rT):
    slab_spec = pl.BlockSpec((8, EB), lambda i: (0, i))
    return pl.pallas_call(
        _dense_body,
        grid=(N_EDGES // EB,),
        in_specs=[
            pl.BlockSpec((NEF, EB), lambda i: (0, i)),
            slab_spec, slab_spec, slab_spec, slab_spec,
            pl.BlockSpec((1, EB), lambda i: (0, i)),
            pl.BlockSpec((NEF, NEF), lambda i: (0, 0)),
            pl.BlockSpec((HID, 1), lambda i: (0, 0)),
            pl.BlockSpec((HID * OUT_DIM, IN_DIM), lambda i: (0, 0)),
            pl.BlockSpec((OUT_DIM, IN_DIM), lambda i: (0, 0)),
        ],
        out_specs=[slab_spec] * 4,
        out_shape=[jax.ShapeDtypeStruct((8, N_EDGES), jnp.float32)] * 4,
    )(eaT, *xs, shT, w1, b1c, w2mT, b2rT)


def _final_body(p_ref, c_ref, na_ref, out_ref):
    s = p_ref[0] + p_ref[1]
    cnt = c_ref[0, :, 0:1] + c_ref[1, :, 0:1]
    out_ref[...] = s / jnp.maximum(cnt, EPS) + na_ref[...]


def _final(psum, pcnt, node_attr):
    return pl.pallas_call(
        _final_body,
        out_shape=jax.ShapeDtypeStruct((N_NODES, OUT_DIM), jnp.float32),
    )(psum, pcnt, node_attr)


def kernel(node_attr, edge_index, edge_attr, edge_sh, fc_w1, fc_b1, fc_w2,
           fc_b2):
    src = edge_index[0]
    dst = edge_index[1]
    xs = _sc_gather()(node_attr, dst)
    b1c = fc_b1.reshape(HID, 1)
    w2mT = ALPHA * fc_w2.reshape(IN_DIM, OUT_DIM, HID).transpose(2, 1, 0)\
        .reshape(HID * OUT_DIM, IN_DIM)
    b2rT = ALPHA * fc_b2.reshape(IN_DIM, OUT_DIM).T
    ss = _dense(edge_attr.T, xs, edge_sh.T, fc_w1, b1c, w2mT, b2rT)
    z32 = jnp.zeros((N_NODES, OUT_DIM), jnp.float32)
    z16 = jnp.zeros((N_NODES, HID), jnp.float32)
    ones = jnp.ones((GC, HID), jnp.float32)
    psum, pcnt = _sc_scatter()(*ss, src, z32, z16, ones)
    return _final(psum, pcnt, node_attr)
